# baseline (speedup 1.0000x reference)
import functools
import math

import jax
import jax.numpy as jnp
from jax import lax
from jax.experimental import pallas as pl
from jax.experimental.pallas import tpu as pltpu
from jax.experimental.pallas import tpu_sc as plsc

N = 10000
E = 320000
HID = 128
HEADS = 8
DK = 16
REL = 256
FF = 512

NWORK = 32
EPW = E // NWORK
STAGE = 32
EPW_PAD = 10240
NSTAGE = EPW_PAD // STAGE
NPAIR = NSTAGE // 2
SGRP = STAGE // 16
NTRASH = 8
NP = N + NTRASH
ROWS_PER_SUB = 624
ROWS_TAIL = N - 16 * ROWS_PER_SUB

_BLK = 1000
_GRID = N // _BLK


def _qkv_body(x_ref, wq_ref, bq_ref, wkv_ref, q_ref, kv_ref):
    xb = x_ref[...]
    q_ref[...] = (
        jnp.dot(xb, wq_ref[...], preferred_element_type=jnp.float32)
        + bq_ref[...]
    )
    kv_ref[...] = jnp.dot(xb, wkv_ref[...], preferred_element_type=jnp.float32)


def _qkv(x, Wq, bq, Wkv):
    return pl.pallas_call(
        _qkv_body,
        grid=(_GRID,),
        in_specs=[
            pl.BlockSpec((_BLK, HID), lambda i: (i, 0)),
            pl.BlockSpec((HID, HID), lambda i: (0, 0)),
            pl.BlockSpec((1, HID), lambda i: (0, 0)),
            pl.BlockSpec((HID, 2 * HID), lambda i: (0, 0)),
        ],
        out_specs=[
            pl.BlockSpec((_BLK, HID), lambda i: (i, 0)),
            pl.BlockSpec((_BLK, 2 * HID), lambda i: (i, 0)),
        ],
        out_shape=[
            jax.ShapeDtypeStruct((N, HID), jnp.float32),
            jax.ShapeDtypeStruct((N, 2 * HID), jnp.float32),
        ],
    )(x, Wq, bq, Wkv)


def _edge_body(
    kv_hbm, q_hbm, rel_hbm, idx_hbm, zwv_hbm, zz_hbm,
    lg_out, wv_out, z_out,
    idxs, kvs, qs, lgs_b, msgs, scos, tp_b,
    acc_wv, acc_z, semg, seml, sems,
):
    c = lax.axis_index("c")
    s = lax.axis_index("s")
    wid = s * 2 + c

    pltpu.sync_copy(zwv_hbm, acc_wv.at[pl.ds(s * ROWS_PER_SUB, ROWS_PER_SUB)])
    pltpu.sync_copy(zz_hbm, acc_z.at[pl.ds(s * ROWS_PER_SUB, ROWS_PER_SUB)])

    @pl.when(s == 15)
    def _zero_tail():
        pltpu.sync_copy(
            zwv_hbm.at[pl.ds(0, ROWS_TAIL)],
            acc_wv.at[pl.ds(16 * ROWS_PER_SUB, ROWS_TAIL)],
        )
        pltpu.sync_copy(
            zz_hbm.at[pl.ds(0, ROWS_TAIL)],
            acc_z.at[pl.ds(16 * ROWS_PER_SUB, ROWS_TAIL)],
        )
    plsc.subcore_barrier()

    iota = lax.broadcasted_iota(jnp.int32, (16,), 0)

    def issue_stage(b, k):
        pltpu.sync_copy(idx_hbm.at[wid, k], idxs[b])
        pltpu.async_copy(kv_hbm.at[idxs[b].at[0]], kvs[b], semg[b])
        pltpu.async_copy(q_hbm.at[idxs[b].at[1]], qs[b], semg[b])
        pltpu.async_copy(rel_hbm.at[idxs[b].at[2]], lgs_b[b], semg[b])

    def drain_gathers(b):
        pltpu.make_async_copy(kv_hbm.at[idxs[b].at[0]], kvs[b], semg[b]).wait()
        pltpu.make_async_copy(q_hbm.at[idxs[b].at[1]], qs[b], semg[b]).wait()
        pltpu.make_async_copy(rel_hbm.at[idxs[b].at[2]], lgs_b[b], semg[b]).wait()

    def compute_stage(b):
        kv_b, q_b, lg_b, msg_b, sco_b = kvs[b], qs[b], lgs_b[b], msgs[b], scos[b]

        def group_step(g, gcarry):
            base = g * 16
            rows = base + iota
            lgrows = [lg_b[base + e, :] for e in range(16)]
            ss = []
            for h in range(HEADS):
                for e in range(16):
                    ke = kv_b[base + e, pl.ds(h * DK, DK)]
                    qe = q_b[base + e, pl.ds(h * DK, DK)]
                    tp_b[e, pl.ds(0, DK)] = (ke + lgrows[e]) * qe
                acc = jnp.zeros((16,), jnp.float32)
                for d in range(DK):
                    acc = acc + plsc.load_gather(
                        tp_b, [iota, jnp.full((16,), d, jnp.int32)]
                    )
                s_h = jnp.exp(jnp.clip(acc * 0.25, -10.0, 10.0))
                ss.append(s_h)
                plsc.store_scatter(
                    sco_b, [rows, jnp.full((16,), h, jnp.int32)], s_h
                )
            for e in range(16):
                for h in range(HEADS):
                    ve = kv_b[base + e, pl.ds(HID + h * DK, DK)]
                    msg_b[base + e, pl.ds(h * DK, DK)] = (
                        (ve + lgrows[e]) * ss[h][e]
                    )
            return gcarry

        lax.fori_loop(0, SGRP, group_step, 0)

    def half(b, k, last):
        drain_gathers(b)
        lgw = pltpu.async_copy(lgs_b[b], lg_out.at[wid * NSTAGE + k], seml[b])
        compute_stage(b)
        pltpu.async_copy(msgs[b], acc_wv.at[idxs[b].at[1]], sems[b], add=True)
        pltpu.async_copy(scos[b], acc_z.at[idxs[b].at[1]], sems[b], add=True)
        lgw.wait()
        pltpu.make_async_copy(msgs[b], acc_wv.at[idxs[b].at[1]], sems[b]).wait()
        pltpu.make_async_copy(scos[b], acc_z.at[idxs[b].at[1]], sems[b]).wait()
        @pl.when(jnp.logical_not(last))
        def _prefetch():
            issue_stage(b, k + 2)

    issue_stage(0, 0)

    def pair_step(m, carry):
        issue_stage(1, 2 * m + 1)
        half(0, 2 * m, m == NPAIR - 1)
        half(1, 2 * m + 1, True)
        return carry

    lax.fori_loop(0, NPAIR, pair_step, 0)
    plsc.subcore_barrier()
    rs = pl.ds(s * ROWS_PER_SUB, ROWS_PER_SUB)
    pltpu.sync_copy(acc_wv.at[rs], wv_out.at[c, rs])
    pltpu.sync_copy(acc_z.at[rs], z_out.at[c, rs])

    @pl.when(s == 15)
    def _dump_tail():
        rt = pl.ds(16 * ROWS_PER_SUB, ROWS_TAIL)
        pltpu.sync_copy(acc_wv.at[rt], wv_out.at[c, rt])
        pltpu.sync_copy(acc_z.at[rt], z_out.at[c, rt])


def _edge(kv_tab, q_tab, rel_table, idx_pack, zwv, zz):
    mesh = plsc.VectorSubcoreMesh(core_axis_name="c", subcore_axis_name="s")

    def body(kv_hbm, q_hbm, rel_hbm, idx_hbm, zwv_hbm,
             zz_hbm, lg_out, wv_out, z_out,
             idx0, idx1, kv0, kv1, q0, q1,
             lg0, lg1, msg0, msg1, sco0, sco1, tp_b, acc_wv, acc_z,
             semg0, semg1, seml0, seml1, sems0, sems1):
        _edge_body(
            kv_hbm, q_hbm, rel_hbm, idx_hbm, zwv_hbm,
            zz_hbm, lg_out, wv_out, z_out,
            (idx0, idx1), (kv0, kv1),
            (q0, q1), (lg0, lg1), (msg0, msg1), (sco0, sco1), tp_b,
            acc_wv, acc_z, (semg0, semg1), (seml0, seml1), (sems0, sems1),
        )

    idx_t = pltpu.VMEM((3, STAGE), jnp.int32)
    f = pl.kernel(
        body,
        out_type=[
            jax.ShapeDtypeStruct((NWORK * NSTAGE, STAGE, DK), jnp.float32),
            jax.ShapeDtypeStruct((2, N, HID), jnp.float32),
            jax.ShapeDtypeStruct((2, N, HEADS), jnp.float32),
        ],
        mesh=mesh,
        compiler_params=pltpu.CompilerParams(
            needs_layout_passes=False, use_tc_tiling_on_sc=False
        ),
        scratch_types=[
            idx_t, idx_t,
            pltpu.VMEM((STAGE, 2 * HID), jnp.float32),
            pltpu.VMEM((STAGE, 2 * HID), jnp.float32),
            pltpu.VMEM((STAGE, HID), jnp.float32),
            pltpu.VMEM((STAGE, HID), jnp.float32),
            pltpu.VMEM((STAGE, DK), jnp.float32),
            pltpu.VMEM((STAGE, DK), jnp.float32),
            pltpu.VMEM((STAGE, HID), jnp.float32),
            pltpu.VMEM((STAGE, HID), jnp.float32),
            pltpu.VMEM((STAGE, HEADS), jnp.float32),
            pltpu.VMEM((STAGE, HEADS), jnp.float32),
            pltpu.VMEM((16, DK + 1), jnp.float32),
            pltpu.VMEM_SHARED((NP, HID), jnp.float32),
            pltpu.VMEM_SHARED((NP, HEADS), jnp.float32),
            pltpu.SemaphoreType.DMA,
            pltpu.SemaphoreType.DMA,
            pltpu.SemaphoreType.DMA,
            pltpu.SemaphoreType.DMA,
            pltpu.SemaphoreType.DMA,
            pltpu.SemaphoreType.DMA,
        ],
    )
    return f(kv_tab, q_tab, rel_table, idx_pack, zwv, zz)


def _post_body(
    wv_ref, z_ref, x_ref, r_ref, wo_ref, bo_ref, w1_ref, b1_ref, w2_ref,
    b2_ref, g1_ref, be1_ref, g2_ref, be2_ref, out_ref,
):
    acc = wv_ref[0] + wv_ref[1]
    zz = z_ref[0] + z_ref[1]
    zrep = jnp.dot(1.0 / zz, r_ref[...], preferred_element_type=jnp.float32)
    o = acc * zrep
    h1 = (
        x_ref[...]
        + jnp.dot(o, wo_ref[...], preferred_element_type=jnp.float32)
        + bo_ref[...]
    )
    m1 = jnp.mean(h1, axis=1, keepdims=True)
    v1 = jnp.mean((h1 - m1) ** 2, axis=1, keepdims=True)
    out1 = g1_ref[...] * (h1 - m1) / jnp.sqrt(v1 + 1e-5) + be1_ref[...]
    ff = jnp.maximum(
        jnp.dot(out1, w1_ref[...], preferred_element_type=jnp.float32)
        + b1_ref[...],
        0.0,
    )
    h2 = (
        out1
        + jnp.dot(ff, w2_ref[...], preferred_element_type=jnp.float32)
        + b2_ref[...]
    )
    m2 = jnp.mean(h2, axis=1, keepdims=True)
    v2 = jnp.mean((h2 - m2) ** 2, axis=1, keepdims=True)
    out_ref[...] = g2_ref[...] * (h2 - m2) / jnp.sqrt(v2 + 1e-5) + be2_ref[...]


def _post(wv2, z2, x, R, Wo, bo, W1, b1, W2, b2, g1, be1, g2, be2):
    full = lambda shape: pl.BlockSpec(shape, lambda i, _s=shape: tuple(0 for _ in _s))
    return pl.pallas_call(
        _post_body,
        grid=(_GRID,),
        in_specs=[
            pl.BlockSpec((2, _BLK, HID), lambda i: (0, i, 0)),
            pl.BlockSpec((2, _BLK, HEADS), lambda i: (0, i, 0)),
            pl.BlockSpec((_BLK, HID), lambda i: (i, 0)),
            full((HEADS, HID)),
            full((HID, HID)),
            full((1, HID)),
            full((HID, FF)),
            full((1, FF)),
            full((FF, HID)),
            full((1, HID)),
            full((1, HID)),
            full((1, HID)),
            full((1, HID)),
            full((1, HID)),
        ],
        out_specs=pl.BlockSpec((_BLK, HID), lambda i: (i, 0)),
        out_shape=jax.ShapeDtypeStruct((N, HID), jnp.float32),
    )(wv2, z2, x, R, Wo, bo, W1, b1, W2, b2, g1, be1, g2, be2)


def kernel(x, edge_index, edge_feat, rel_table, Wq, bq, Wk, Wv, Wo, bo,
           W1, b1, W2, b2, ln1_g, ln1_b, ln2_g, ln2_b):
    ei = edge_index.astype(jnp.int32)
    ef = edge_feat.astype(jnp.int32)
    npad = EPW_PAD - EPW
    src3 = jnp.pad(ei[0].reshape(NWORK, EPW), ((0, 0), (0, npad))).reshape(
        NWORK, NSTAGE, STAGE
    )
    dpad = N + (jnp.arange(npad, dtype=jnp.int32) % NTRASH)
    dst3 = jnp.concatenate(
        [ei[1].reshape(NWORK, EPW), jnp.broadcast_to(dpad, (NWORK, npad))],
        axis=1,
    ).reshape(NWORK, NSTAGE, STAGE)
    feat3 = jnp.pad(ef.reshape(NWORK, EPW), ((0, 0), (0, npad))).reshape(
        NWORK, NSTAGE, STAGE
    )
    idx_pack = jnp.stack([src3, dst3, feat3], axis=2)
    Wkv = jnp.concatenate([Wk, Wv], axis=1)

    q_tab, kv_tab = _qkv(x, Wq, bq.reshape(1, HID), Wkv)

    zwv = jnp.zeros((ROWS_PER_SUB, HID), jnp.float32)
    zz = jnp.zeros((ROWS_PER_SUB, HEADS), jnp.float32)
    lg4, wv2, z2 = _edge(kv_tab, q_tab, rel_table, idx_pack, zwv, zz)
    lg_x = lg4.reshape(NWORK, EPW_PAD, DK)[:, :EPW, :].reshape(E, DK)

    R = jnp.repeat(jnp.eye(HEADS, dtype=jnp.float32), DK, axis=1)
    out_x = _post(
        wv2, z2, x, R, Wo, bo.reshape(1, HID), W1, b1.reshape(1, FF),
        W2, b2.reshape(1, HID), ln1_g.reshape(1, HID), ln1_b.reshape(1, HID),
        ln2_g.reshape(1, HID), ln2_b.reshape(1, HID),
    )
    return (out_x, lg_x)


# speedup vs baseline: 1.0505x; 1.0505x over previous
import functools
import math

import jax
import jax.numpy as jnp
from jax import lax
from jax.experimental import pallas as pl
from jax.experimental.pallas import tpu as pltpu
from jax.experimental.pallas import tpu_sc as plsc

N = 10000
E = 320000
HID = 128
HEADS = 8
DK = 16
REL = 256
FF = 512

NWORK = 32
EPW = E // NWORK
CHUNK = 80
NCHUNK = EPW // CHUNK
GROUPS = CHUNK // 16
ROWS_PER_SUB = 624
ROWS_TAIL = N - 16 * ROWS_PER_SUB

_BLK = 1000
_GRID = N // _BLK


def _qkv_body(x_ref, wq_ref, bq_ref, wkv_ref, q_ref, kv_ref):
    xb = x_ref[...]
    q_ref[...] = (
        jnp.dot(xb, wq_ref[...], preferred_element_type=jnp.float32)
        + bq_ref[...]
    )
    kv_ref[...] = jnp.dot(xb, wkv_ref[...], preferred_element_type=jnp.float32)


def _qkv(x, Wq, bq, Wkv):
    return pl.pallas_call(
        _qkv_body,
        grid=(_GRID,),
        in_specs=[
            pl.BlockSpec((_BLK, HID), lambda i: (i, 0)),
            pl.BlockSpec((HID, HID), lambda i: (0, 0)),
            pl.BlockSpec((1, HID), lambda i: (0, 0)),
            pl.BlockSpec((HID, 2 * HID), lambda i: (0, 0)),
        ],
        out_specs=[
            pl.BlockSpec((_BLK, HID), lambda i: (i, 0)),
            pl.BlockSpec((_BLK, 2 * HID), lambda i: (i, 0)),
        ],
        out_shape=[
            jax.ShapeDtypeStruct((N, HID), jnp.float32),
            jax.ShapeDtypeStruct((N, 2 * HID), jnp.float32),
        ],
    )(x, Wq, bq, Wkv)


def _edge_body(
    kv_hbm, q_hbm, rel_hbm, idx_hbm, zwv_hbm, zz_hbm,
    lg_out, wv_out, z_out,
    idx0, idx1, kv_b, q_b, lg_b, msg_b, sco_b, tp_b,
    acc_wv, acc_z, semg, seml, sems,
):
    c = lax.axis_index("c")
    s = lax.axis_index("s")
    wid = s * 2 + c

    pltpu.sync_copy(zwv_hbm, acc_wv.at[pl.ds(s * ROWS_PER_SUB, ROWS_PER_SUB)])
    pltpu.sync_copy(zz_hbm, acc_z.at[pl.ds(s * ROWS_PER_SUB, ROWS_PER_SUB)])

    @pl.when(s == 15)
    def _zero_tail():
        pltpu.sync_copy(
            zwv_hbm.at[pl.ds(0, ROWS_TAIL)],
            acc_wv.at[pl.ds(16 * ROWS_PER_SUB, ROWS_TAIL)],
        )
        pltpu.sync_copy(
            zz_hbm.at[pl.ds(0, ROWS_TAIL)],
            acc_z.at[pl.ds(16 * ROWS_PER_SUB, ROWS_TAIL)],
        )
    plsc.subcore_barrier()

    iota = lax.broadcasted_iota(jnp.int32, (16,), 0)

    def chunk_step(j, idx_b, first):
        pltpu.sync_copy(idx_hbm.at[wid, j], idx_b)

        @pl.when(jnp.logical_not(first))
        def _drain_prev_lg():
            pltpu.make_async_copy(lg_b, lg_out.at[0], seml).wait()

        cp0 = pltpu.async_copy(kv_hbm.at[idx_b.at[0]], kv_b, semg)
        cp1 = pltpu.async_copy(q_hbm.at[idx_b.at[1]], q_b, semg)
        cp2 = pltpu.async_copy(rel_hbm.at[idx_b.at[2]], lg_b, semg)

        @pl.when(jnp.logical_not(first))
        def _drain_prev_scatter():
            pltpu.make_async_copy(msg_b, acc_wv.at[idx_b.at[1]], sems).wait()
            pltpu.make_async_copy(sco_b, acc_z.at[idx_b.at[1]], sems).wait()
        cp0.wait()
        cp1.wait()
        cp2.wait()
        pltpu.async_copy(lg_b, lg_out.at[wid * NCHUNK + j], seml)

        def group_step(g, gcarry):
            base = g * 16
            rows = base + iota
            lgs = [lg_b[base + e, :] for e in range(16)]
            ss = []
            for h in range(HEADS):
                for e in range(16):
                    ke = kv_b[base + e, pl.ds(h * DK, DK)]
                    qe = q_b[base + e, pl.ds(h * DK, DK)]
                    tp_b[e, pl.ds(0, DK)] = (ke + lgs[e]) * qe
                acc = jnp.zeros((16,), jnp.float32)
                for d in range(DK):
                    acc = acc + plsc.load_gather(
                        tp_b, [iota, jnp.full((16,), d, jnp.int32)]
                    )
                s_h = jnp.exp(jnp.clip(acc * 0.25, -10.0, 10.0))
                ss.append(s_h)
                plsc.store_scatter(
                    sco_b, [rows, jnp.full((16,), h, jnp.int32)], s_h
                )
            for e in range(16):
                for h in range(HEADS):
                    ve = kv_b[base + e, pl.ds(HID + h * DK, DK)]
                    msg_b[base + e, pl.ds(h * DK, DK)] = (ve + lgs[e]) * ss[h][e]
            return gcarry

        lax.fori_loop(0, GROUPS, group_step, 0)
        pltpu.async_copy(msg_b, acc_wv.at[idx_b.at[1]], sems, add=True)
        pltpu.async_copy(sco_b, acc_z.at[idx_b.at[1]], sems, add=True)

    def pair_step(m, carry):
        chunk_step(2 * m, idx0, m == 0)
        chunk_step(2 * m + 1, idx1, False)
        return carry

    lax.fori_loop(0, NCHUNK // 2, pair_step, 0)
    chunk_step(NCHUNK - 1, idx0, False)
    pltpu.make_async_copy(msg_b, acc_wv.at[idx0.at[1]], sems).wait()
    pltpu.make_async_copy(sco_b, acc_z.at[idx0.at[1]], sems).wait()
    pltpu.make_async_copy(lg_b, lg_out.at[0], seml).wait()
    plsc.subcore_barrier()
    rs = pl.ds(s * ROWS_PER_SUB, ROWS_PER_SUB)
    pltpu.sync_copy(acc_wv.at[rs], wv_out.at[c, rs])
    pltpu.sync_copy(acc_z.at[rs], z_out.at[c, rs])

    @pl.when(s == 15)
    def _dump_tail():
        rt = pl.ds(16 * ROWS_PER_SUB, ROWS_TAIL)
        pltpu.sync_copy(acc_wv.at[rt], wv_out.at[c, rt])
        pltpu.sync_copy(acc_z.at[rt], z_out.at[c, rt])


def _edge(kv_tab, q_tab, rel_table, idx_pack, zwv, zz):
    mesh = plsc.VectorSubcoreMesh(core_axis_name="c", subcore_axis_name="s")
    f = pl.kernel(
        _edge_body,
        out_type=[
            jax.ShapeDtypeStruct((NWORK * NCHUNK, CHUNK, DK), jnp.float32),
            jax.ShapeDtypeStruct((2, N, HID), jnp.float32),
            jax.ShapeDtypeStruct((2, N, HEADS), jnp.float32),
        ],
        mesh=mesh,
        compiler_params=pltpu.CompilerParams(
            needs_layout_passes=False, use_tc_tiling_on_sc=False
        ),
        scratch_types=[
            pltpu.VMEM((3, CHUNK), jnp.int32),
            pltpu.VMEM((3, CHUNK), jnp.int32),
            pltpu.VMEM((CHUNK, 2 * HID), jnp.float32),
            pltpu.VMEM((CHUNK, HID), jnp.float32),
            pltpu.VMEM((CHUNK, DK), jnp.float32),
            pltpu.VMEM((CHUNK, HID), jnp.float32),
            pltpu.VMEM((CHUNK, HEADS), jnp.float32),
            pltpu.VMEM((16, DK + 1), jnp.float32),
            pltpu.VMEM_SHARED((N, HID), jnp.float32),
            pltpu.VMEM_SHARED((N, HEADS), jnp.float32),
            pltpu.SemaphoreType.DMA,
            pltpu.SemaphoreType.DMA,
            pltpu.SemaphoreType.DMA,
        ],
    )
    return f(kv_tab, q_tab, rel_table, idx_pack, zwv, zz)


def _post_body(
    wv_ref, z_ref, x_ref, r_ref, wo_ref, bo_ref, w1_ref, b1_ref, w2_ref,
    b2_ref, g1_ref, be1_ref, g2_ref, be2_ref, out_ref,
):
    acc = wv_ref[0] + wv_ref[1]
    zz = z_ref[0] + z_ref[1]
    zrep = jnp.dot(1.0 / zz, r_ref[...], preferred_element_type=jnp.float32)
    o = acc * zrep
    h1 = (
        x_ref[...]
        + jnp.dot(o, wo_ref[...], preferred_element_type=jnp.float32)
        + bo_ref[...]
    )
    m1 = jnp.mean(h1, axis=1, keepdims=True)
    v1 = jnp.mean((h1 - m1) ** 2, axis=1, keepdims=True)
    out1 = g1_ref[...] * (h1 - m1) / jnp.sqrt(v1 + 1e-5) + be1_ref[...]
    ff = jnp.maximum(
        jnp.dot(out1, w1_ref[...], preferred_element_type=jnp.float32)
        + b1_ref[...],
        0.0,
    )
    h2 = (
        out1
        + jnp.dot(ff, w2_ref[...], preferred_element_type=jnp.float32)
        + b2_ref[...]
    )
    m2 = jnp.mean(h2, axis=1, keepdims=True)
    v2 = jnp.mean((h2 - m2) ** 2, axis=1, keepdims=True)
    out_ref[...] = g2_ref[...] * (h2 - m2) / jnp.sqrt(v2 + 1e-5) + be2_ref[...]


def _post(wv2, z2, x, R, Wo, bo, W1, b1, W2, b2, g1, be1, g2, be2):
    full = lambda shape: pl.BlockSpec(shape, lambda i, _s=shape: tuple(0 for _ in _s))
    return pl.pallas_call(
        _post_body,
        grid=(_GRID,),
        in_specs=[
            pl.BlockSpec((2, _BLK, HID), lambda i: (0, i, 0)),
            pl.BlockSpec((2, _BLK, HEADS), lambda i: (0, i, 0)),
            pl.BlockSpec((_BLK, HID), lambda i: (i, 0)),
            full((HEADS, HID)),
            full((HID, HID)),
            full((1, HID)),
            full((HID, FF)),
            full((1, FF)),
            full((FF, HID)),
            full((1, HID)),
            full((1, HID)),
            full((1, HID)),
            full((1, HID)),
            full((1, HID)),
        ],
        out_specs=pl.BlockSpec((_BLK, HID), lambda i: (i, 0)),
        out_shape=jax.ShapeDtypeStruct((N, HID), jnp.float32),
    )(wv2, z2, x, R, Wo, bo, W1, b1, W2, b2, g1, be1, g2, be2)


def kernel(x, edge_index, edge_feat, rel_table, Wq, bq, Wk, Wv, Wo, bo,
           W1, b1, W2, b2, ln1_g, ln1_b, ln2_g, ln2_b):
    ei = edge_index.astype(jnp.int32)
    ef = edge_feat.astype(jnp.int32)
    src3 = ei[0].reshape(NWORK, NCHUNK, CHUNK)
    dst3 = ei[1].reshape(NWORK, NCHUNK, CHUNK)
    feat3 = ef.reshape(NWORK, NCHUNK, CHUNK)
    idx_pack = jnp.stack([src3, dst3, feat3], axis=2)
    Wkv = jnp.concatenate([Wk, Wv], axis=1)

    q_tab, kv_tab = _qkv(x, Wq, bq.reshape(1, HID), Wkv)

    zwv = jnp.zeros((ROWS_PER_SUB, HID), jnp.float32)
    zz = jnp.zeros((ROWS_PER_SUB, HEADS), jnp.float32)
    lg4, wv2, z2 = _edge(kv_tab, q_tab, rel_table, idx_pack, zwv, zz)
    lg_x = lg4.reshape(E, DK)

    R = jnp.repeat(jnp.eye(HEADS, dtype=jnp.float32), DK, axis=1)
    out_x = _post(
        wv2, z2, x, R, Wo, bo.reshape(1, HID), W1, b1.reshape(1, FF),
        W2, b2.reshape(1, HID), ln1_g.reshape(1, HID), ln1_b.reshape(1, HID),
        ln2_g.reshape(1, HID), ln2_b.reshape(1, HID),
    )
    return (out_x, lg_x)


# speedup vs baseline: 1.1231x; 1.0691x over previous
import functools
import math

import jax
import jax.numpy as jnp
from jax import lax
from jax.experimental import pallas as pl
from jax.experimental.pallas import tpu as pltpu
from jax.experimental.pallas import tpu_sc as plsc

N = 10000
E = 320000
HID = 128
HEADS = 8
DK = 16
REL = 256
FF = 512

NWORK = 32
EPW = E // NWORK
CHUNK = 80
NCHUNK = EPW // CHUNK
GROUPS = CHUNK // 16
ROWS_PER_SUB = 624
ROWS_TAIL = N - 16 * ROWS_PER_SUB

_BLK = 1000
_GRID = N // _BLK


def _qkv_body(x_ref, wq_ref, bq_ref, wkv_ref, q_ref, kv_ref):
    xb = x_ref[...]
    q_ref[...] = (
        jnp.dot(xb, wq_ref[...], preferred_element_type=jnp.float32)
        + bq_ref[...]
    )
    kv_ref[...] = jnp.dot(xb, wkv_ref[...], preferred_element_type=jnp.float32)


def _qkv(x, Wq, bq, Wkv):
    return pl.pallas_call(
        _qkv_body,
        grid=(_GRID,),
        in_specs=[
            pl.BlockSpec((_BLK, HID), lambda i: (i, 0)),
            pl.BlockSpec((HID, HID), lambda i: (0, 0)),
            pl.BlockSpec((1, HID), lambda i: (0, 0)),
            pl.BlockSpec((HID, 2 * HID), lambda i: (0, 0)),
        ],
        out_specs=[
            pl.BlockSpec((_BLK, HID), lambda i: (i, 0)),
            pl.BlockSpec((_BLK, 2 * HID), lambda i: (i, 0)),
        ],
        out_shape=[
            jax.ShapeDtypeStruct((N, HID), jnp.float32),
            jax.ShapeDtypeStruct((N, 2 * HID), jnp.float32),
        ],
    )(x, Wq, bq, Wkv)


def _edge_body(
    kv_hbm, q_hbm, rel_hbm, src_hbm, dst_hbm, feat_hbm, zwv_hbm, zz_hbm,
    lg_out, wv_out, z_out,
    src_b, dst_b, feat_b, kv_b, q_b, lg_b, msg_b, sco_b, tp_b,
    acc_wv, acc_z, sem0, sem1, sem2,
):
    c = lax.axis_index("c")
    s = lax.axis_index("s")
    wid = s * 2 + c

    pltpu.sync_copy(zwv_hbm, acc_wv.at[pl.ds(s * ROWS_PER_SUB, ROWS_PER_SUB)])
    pltpu.sync_copy(zz_hbm, acc_z.at[pl.ds(s * ROWS_PER_SUB, ROWS_PER_SUB)])

    @pl.when(s == 15)
    def _zero_tail():
        pltpu.sync_copy(
            zwv_hbm.at[pl.ds(0, ROWS_TAIL)],
            acc_wv.at[pl.ds(16 * ROWS_PER_SUB, ROWS_TAIL)],
        )
        pltpu.sync_copy(
            zz_hbm.at[pl.ds(0, ROWS_TAIL)],
            acc_z.at[pl.ds(16 * ROWS_PER_SUB, ROWS_TAIL)],
        )
    plsc.subcore_barrier()

    iota = lax.broadcasted_iota(jnp.int32, (16,), 0)

    def chunk_step(j, carry):
        pltpu.sync_copy(src_hbm.at[wid, j], src_b)
        pltpu.sync_copy(dst_hbm.at[wid, j], dst_b)
        pltpu.sync_copy(feat_hbm.at[wid, j], feat_b)
        cp0 = pltpu.async_copy(kv_hbm.at[src_b], kv_b, sem0)
        cp1 = pltpu.async_copy(q_hbm.at[dst_b], q_b, sem1)
        cp2 = pltpu.async_copy(rel_hbm.at[feat_b], lg_b, sem2)
        cp0.wait()
        cp1.wait()
        cp2.wait()
        pltpu.sync_copy(lg_b, lg_out.at[wid * NCHUNK + j])

        def group_step(g, gcarry):
            base = g * 16
            rows = base + iota
            lgs = [lg_b[base + e, :] for e in range(16)]
            ss = []
            for h in range(HEADS):
                for e in range(16):
                    ke = kv_b[base + e, pl.ds(h * DK, DK)]
                    qe = q_b[base + e, pl.ds(h * DK, DK)]
                    tp_b[e, pl.ds(0, DK)] = (ke + lgs[e]) * qe
                acc = jnp.zeros((16,), jnp.float32)
                for d in range(DK):
                    acc = acc + plsc.load_gather(
                        tp_b, [iota, jnp.full((16,), d, jnp.int32)]
                    )
                s_h = jnp.exp(jnp.clip(acc * 0.25, -10.0, 10.0))
                ss.append(s_h)
                plsc.store_scatter(
                    sco_b, [rows, jnp.full((16,), h, jnp.int32)], s_h
                )
            for e in range(16):
                for h in range(HEADS):
                    ve = kv_b[base + e, pl.ds(HID + h * DK, DK)]
                    msg_b[base + e, pl.ds(h * DK, DK)] = (ve + lgs[e]) * ss[h][e]
            return gcarry

        lax.fori_loop(0, GROUPS, group_step, 0)
        pltpu.sync_copy(msg_b, acc_wv.at[dst_b], add=True)
        pltpu.sync_copy(sco_b, acc_z.at[dst_b], add=True)
        return carry

    lax.fori_loop(0, NCHUNK, chunk_step, 0)
    plsc.subcore_barrier()
    rs = pl.ds(s * ROWS_PER_SUB, ROWS_PER_SUB)
    pltpu.sync_copy(acc_wv.at[rs], wv_out.at[c, rs])
    pltpu.sync_copy(acc_z.at[rs], z_out.at[c, rs])

    @pl.when(s == 15)
    def _dump_tail():
        rt = pl.ds(16 * ROWS_PER_SUB, ROWS_TAIL)
        pltpu.sync_copy(acc_wv.at[rt], wv_out.at[c, rt])
        pltpu.sync_copy(acc_z.at[rt], z_out.at[c, rt])


def _edge(kv_tab, q_tab, rel_table, src3, dst3, feat3, zwv, zz):
    mesh = plsc.VectorSubcoreMesh(core_axis_name="c", subcore_axis_name="s")
    f = pl.kernel(
        _edge_body,
        out_type=[
            jax.ShapeDtypeStruct((NWORK * NCHUNK, CHUNK, DK), jnp.float32),
            jax.ShapeDtypeStruct((2, N, HID), jnp.float32),
            jax.ShapeDtypeStruct((2, N, HEADS), jnp.float32),
        ],
        mesh=mesh,
        compiler_params=pltpu.CompilerParams(
            needs_layout_passes=False, use_tc_tiling_on_sc=False
        ),
        scratch_types=[
            pltpu.VMEM((CHUNK,), jnp.int32),
            pltpu.VMEM((CHUNK,), jnp.int32),
            pltpu.VMEM((CHUNK,), jnp.int32),
            pltpu.VMEM((CHUNK, 2 * HID), jnp.float32),
            pltpu.VMEM((CHUNK, HID), jnp.float32),
            pltpu.VMEM((CHUNK, DK), jnp.float32),
            pltpu.VMEM((CHUNK, HID), jnp.float32),
            pltpu.VMEM((CHUNK, HEADS), jnp.float32),
            pltpu.VMEM((16, DK + 1), jnp.float32),
            pltpu.VMEM_SHARED((N, HID), jnp.float32),
            pltpu.VMEM_SHARED((N, HEADS), jnp.float32),
            pltpu.SemaphoreType.DMA,
            pltpu.SemaphoreType.DMA,
            pltpu.SemaphoreType.DMA,
        ],
    )
    return f(kv_tab, q_tab, rel_table, src3, dst3, feat3, zwv, zz)


def _post_body(
    wv_ref, z_ref, x_ref, r_ref, wo_ref, bo_ref, w1_ref, b1_ref, w2_ref,
    b2_ref, g1_ref, be1_ref, g2_ref, be2_ref, out_ref,
):
    acc = wv_ref[0] + wv_ref[1]
    zz = z_ref[0] + z_ref[1]
    zrep = jnp.dot(1.0 / zz, r_ref[...], preferred_element_type=jnp.float32)
    o = acc * zrep
    h1 = (
        x_ref[...]
        + jnp.dot(o, wo_ref[...], preferred_element_type=jnp.float32)
        + bo_ref[...]
    )
    m1 = jnp.mean(h1, axis=1, keepdims=True)
    v1 = jnp.mean((h1 - m1) ** 2, axis=1, keepdims=True)
    out1 = g1_ref[...] * (h1 - m1) / jnp.sqrt(v1 + 1e-5) + be1_ref[...]
    ff = jnp.maximum(
        jnp.dot(out1, w1_ref[...], preferred_element_type=jnp.float32)
        + b1_ref[...],
        0.0,
    )
    h2 = (
        out1
        + jnp.dot(ff, w2_ref[...], preferred_element_type=jnp.float32)
        + b2_ref[...]
    )
    m2 = jnp.mean(h2, axis=1, keepdims=True)
    v2 = jnp.mean((h2 - m2) ** 2, axis=1, keepdims=True)
    out_ref[...] = g2_ref[...] * (h2 - m2) / jnp.sqrt(v2 + 1e-5) + be2_ref[...]


def _post(wv2, z2, x, R, Wo, bo, W1, b1, W2, b2, g1, be1, g2, be2):
    full = lambda shape: pl.BlockSpec(shape, lambda i, _s=shape: tuple(0 for _ in _s))
    return pl.pallas_call(
        _post_body,
        grid=(_GRID,),
        in_specs=[
            pl.BlockSpec((2, _BLK, HID), lambda i: (0, i, 0)),
            pl.BlockSpec((2, _BLK, HEADS), lambda i: (0, i, 0)),
            pl.BlockSpec((_BLK, HID), lambda i: (i, 0)),
            full((HEADS, HID)),
            full((HID, HID)),
            full((1, HID)),
            full((HID, FF)),
            full((1, FF)),
            full((FF, HID)),
            full((1, HID)),
            full((1, HID)),
            full((1, HID)),
            full((1, HID)),
            full((1, HID)),
        ],
        out_specs=pl.BlockSpec((_BLK, HID), lambda i: (i, 0)),
        out_shape=jax.ShapeDtypeStruct((N, HID), jnp.float32),
    )(wv2, z2, x, R, Wo, bo, W1, b1, W2, b2, g1, be1, g2, be2)


def kernel(x, edge_index, edge_feat, rel_table, Wq, bq, Wk, Wv, Wo, bo,
           W1, b1, W2, b2, ln1_g, ln1_b, ln2_g, ln2_b):
    ei = edge_index.astype(jnp.int32)
    ef = edge_feat.astype(jnp.int32)
    src3 = ei[0].reshape(NWORK, NCHUNK, CHUNK)
    dst3 = ei[1].reshape(NWORK, NCHUNK, CHUNK)
    feat3 = ef.reshape(NWORK, NCHUNK, CHUNK)
    Wkv = jnp.concatenate([Wk, Wv], axis=1)

    q_tab, kv_tab = _qkv(x, Wq, bq.reshape(1, HID), Wkv)

    zwv = jnp.zeros((ROWS_PER_SUB, HID), jnp.float32)
    zz = jnp.zeros((ROWS_PER_SUB, HEADS), jnp.float32)
    lg4, wv2, z2 = _edge(kv_tab, q_tab, rel_table, src3, dst3, feat3, zwv, zz)
    lg_x = lg4.reshape(E, DK)

    R = jnp.repeat(jnp.eye(HEADS, dtype=jnp.float32), DK, axis=1)
    out_x = _post(
        wv2, z2, x, R, Wo, bo.reshape(1, HID), W1, b1.reshape(1, FF),
        W2, b2.reshape(1, HID), ln1_g.reshape(1, HID), ln1_b.reshape(1, HID),
        ln2_g.reshape(1, HID), ln2_b.reshape(1, HID),
    )
    return (out_x, lg_x)
